# initial kernel scaffold (unmeasured)
import jax
import jax.numpy as jnp
from jax import lax
from jax.experimental import pallas as pl
from jax.experimental.pallas import tpu as pltpu

N_DEV = 32
LOG2_N = 5
H_PER = 4
B, SQ, SKV, DH = 2, 128, 128, 64
DM = 512
BLK = 64


def kernel(x, Wq, K_ext, V_ext, Wo):
    pos = lax.axis_index("i")
    k_loc = lax.dynamic_slice_in_dim(K_ext, pos * H_PER, H_PER, axis=2)
    v_loc = lax.dynamic_slice_in_dim(V_ext, pos * H_PER, H_PER, axis=2)
    k_loc = k_loc.transpose(0, 2, 1, 3)
    v_loc = v_loc.transpose(0, 2, 1, 3)

    def body(x_ref, wq_ref, k_ref, v_ref, wo_ref, out_ref,
             recv_ref, send_sems, recv_sems):
        my = lax.axis_index("i")

        xm = x_ref[...].reshape(B * SQ, DM)
        q = jnp.dot(xm, wq_ref[...], preferred_element_type=jnp.float32)

        qi = lax.broadcasted_iota(jnp.int32, (SQ, SKV), 0)
        kj = lax.broadcasted_iota(jnp.int32, (SQ, SKV), 1)
        mask = (kj // BLK) <= (qi // BLK)

        b_rows = []
        for b in range(B):
            h_cols = []
            for h in range(H_PER):
                q_bh = q[b * SQ:(b + 1) * SQ, h * DH:(h + 1) * DH]
                k_bh = k_ref[b, h]
                v_bh = v_ref[b, h]
                s = jnp.dot(q_bh, k_bh.T,
                            preferred_element_type=jnp.float32) * 0.125
                s = jnp.where(mask, s, -1e9)
                m = jnp.max(s, axis=-1, keepdims=True)
                w = jnp.exp(s - m)
                w = w / jnp.sum(w, axis=-1, keepdims=True)
                h_cols.append(jnp.dot(w, v_bh,
                                      preferred_element_type=jnp.float32))
            b_rows.append(jnp.concatenate(h_cols, axis=1))
        ctx = jnp.concatenate(b_rows, axis=0)

        partial = jnp.dot(ctx, wo_ref[...], preferred_element_type=jnp.float32)
        out_ref[...] = partial.reshape(B, SQ, DM)

        for r in range(LOG2_N):
            partner = my ^ (1 << r)
            rdma = pltpu.make_async_remote_copy(
                src_ref=out_ref,
                dst_ref=recv_ref.at[r],
                send_sem=send_sems.at[r],
                recv_sem=recv_sems.at[r],
                device_id=(partner,),
                device_id_type=pl.DeviceIdType.MESH,
            )
            rdma.start()
            rdma.wait()
            out_ref[...] = out_ref[...] + recv_ref[r]

    return pl.pallas_call(
        body,
        out_shape=jax.ShapeDtypeStruct((B, SQ, DM), jnp.float32),
        in_specs=[
            pl.BlockSpec(memory_space=pltpu.VMEM),
            pl.BlockSpec(memory_space=pltpu.VMEM),
            pl.BlockSpec(memory_space=pltpu.VMEM),
            pl.BlockSpec(memory_space=pltpu.VMEM),
            pl.BlockSpec(memory_space=pltpu.VMEM),
        ],
        out_specs=pl.BlockSpec(memory_space=pltpu.VMEM),
        scratch_shapes=[
            pltpu.VMEM((LOG2_N, B, SQ, DM), jnp.float32),
            pltpu.SemaphoreType.DMA((LOG2_N,)),
            pltpu.SemaphoreType.DMA((LOG2_N,)),
        ],
        compiler_params=pltpu.CompilerParams(collective_id=0),
    )(x, Wq, k_loc, v_loc, Wo)


# baseline (device time: 80950 ns/iter reference)
import jax
import jax.numpy as jnp
from jax import lax
from jax.experimental import pallas as pl
from jax.experimental.pallas import tpu as pltpu

N_DEV = 32
LOG2_N = 5
H_PER = 4
B, SQ, SKV, DH = 2, 128, 128, 64
DM = 512
BLK = 64


def kernel(x, Wq, K_ext, V_ext, Wo):
    pos = lax.axis_index("i")
    k_loc = lax.dynamic_slice_in_dim(K_ext, pos * H_PER, H_PER, axis=2)
    v_loc = lax.dynamic_slice_in_dim(V_ext, pos * H_PER, H_PER, axis=2)
    k_loc = k_loc.transpose(0, 2, 1, 3)
    v_loc = v_loc.transpose(0, 2, 1, 3)

    def body(x_ref, wq_ref, k_ref, v_ref, wo_ref, out_ref,
             recv_ref, send_sems, recv_sems):
        my = lax.axis_index("i")

        xm = x_ref[...].reshape(B * SQ, DM)
        q = jnp.dot(xm, wq_ref[...], preferred_element_type=jnp.float32)

        qi = lax.broadcasted_iota(jnp.int32, (SQ, SKV), 0)
        kj = lax.broadcasted_iota(jnp.int32, (SQ, SKV), 1)
        mask = (kj // BLK) <= (qi // BLK)

        b_rows = []
        for b in range(B):
            h_cols = []
            for h in range(H_PER):
                q_bh = q[b * SQ:(b + 1) * SQ, h * DH:(h + 1) * DH]
                k_bh = k_ref[b, h]
                v_bh = v_ref[b, h]
                s = jnp.dot(q_bh, k_bh.T,
                            preferred_element_type=jnp.float32) * 0.125
                s = jnp.where(mask, s, -1e9)
                m = jnp.max(s, axis=-1, keepdims=True)
                w = jnp.exp(s - m)
                w = w / jnp.sum(w, axis=-1, keepdims=True)
                h_cols.append(jnp.dot(w, v_bh,
                                      preferred_element_type=jnp.float32))
            b_rows.append(jnp.concatenate(h_cols, axis=1))
        ctx = jnp.concatenate(b_rows, axis=0)

        partial = jnp.dot(ctx, wo_ref[...], preferred_element_type=jnp.float32)
        out_ref[...] = partial.reshape(B, SQ, DM)

        for r in range(LOG2_N):
            partner = my ^ (1 << r)
            rdma = pltpu.make_async_remote_copy(
                src_ref=out_ref,
                dst_ref=recv_ref.at[r],
                send_sem=send_sems.at[r],
                recv_sem=recv_sems.at[r],
                device_id=(partner,),
                device_id_type=pl.DeviceIdType.MESH,
            )
            rdma.start()
            rdma.wait()
            out_ref[...] = out_ref[...] + recv_ref[r]

    return pl.pallas_call(
        body,
        out_shape=jax.ShapeDtypeStruct((B, SQ, DM), jnp.float32),
        in_specs=[
            pl.BlockSpec(memory_space=pltpu.VMEM),
            pl.BlockSpec(memory_space=pltpu.VMEM),
            pl.BlockSpec(memory_space=pltpu.VMEM),
            pl.BlockSpec(memory_space=pltpu.VMEM),
            pl.BlockSpec(memory_space=pltpu.VMEM),
        ],
        out_specs=pl.BlockSpec(memory_space=pltpu.VMEM),
        scratch_shapes=[
            pltpu.VMEM((LOG2_N, B, SQ, DM), jnp.float32),
            pltpu.SemaphoreType.DMA((LOG2_N,)),
            pltpu.SemaphoreType.DMA((LOG2_N,)),
        ],
    )(x, Wq, k_loc, v_loc, Wo)


# device time: 18854 ns/iter; 4.2935x vs baseline; 4.2935x over previous
import jax
import jax.numpy as jnp
from jax import lax
from jax.experimental import pallas as pl
from jax.experimental.pallas import tpu as pltpu

N_DEV = 32
LOG2_N = 5
H_PER = 4
B, SQ, SKV, DH = 2, 128, 128, 64
DM = 512
BLK = 64
ROWS = B * SQ


def kernel(x, Wq, K_ext, V_ext, Wo):
    pos = lax.axis_index("i")
    k_loc = lax.dynamic_slice_in_dim(K_ext, pos * H_PER, H_PER, axis=2)
    v_loc = lax.dynamic_slice_in_dim(V_ext, pos * H_PER, H_PER, axis=2)
    k_loc = k_loc.transpose(0, 2, 1, 3)
    v_loc = v_loc.transpose(0, 2, 1, 3)

    def body(x_ref, wq_ref, k_ref, v_ref, wo_ref, out_ref,
             acc_ref, recv_ref, send_sems, recv_sems):
        my = lax.axis_index("i")

        xm = x_ref[...].reshape(ROWS, DM)
        q = jnp.dot(xm, wq_ref[...], preferred_element_type=jnp.float32)

        qi = lax.broadcasted_iota(jnp.int32, (SQ, SKV), 0)
        kj = lax.broadcasted_iota(jnp.int32, (SQ, SKV), 1)
        mask = (kj // BLK) <= (qi // BLK)

        b_rows = []
        for b in range(B):
            h_cols = []
            for h in range(H_PER):
                q_bh = q[b * SQ:(b + 1) * SQ, h * DH:(h + 1) * DH]
                k_bh = k_ref[b, h]
                v_bh = v_ref[b, h]
                s = jnp.dot(q_bh, k_bh.T,
                            preferred_element_type=jnp.float32) * 0.125
                s = jnp.where(mask, s, -1e9)
                m = jnp.max(s, axis=-1, keepdims=True)
                w = jnp.exp(s - m)
                w = w / jnp.sum(w, axis=-1, keepdims=True)
                h_cols.append(jnp.dot(w, v_bh,
                                      preferred_element_type=jnp.float32))
            b_rows.append(jnp.concatenate(h_cols, axis=1))
        ctx = jnp.concatenate(b_rows, axis=0)

        acc_ref[...] = jnp.dot(ctx, wo_ref[...],
                               preferred_element_type=jnp.float32)

        base = jnp.int32(0)
        for r in range(LOG2_N):
            half = ROWS >> (r + 1)
            bit = (my >> r) & 1
            keep_base = base + bit * half
            send_base = base + (1 - bit) * half
            partner = my ^ (1 << r)
            rdma = pltpu.make_async_remote_copy(
                src_ref=acc_ref.at[pl.ds(send_base, half)],
                dst_ref=recv_ref.at[r, pl.ds(0, half)],
                send_sem=send_sems.at[r],
                recv_sem=recv_sems.at[r],
                device_id=(partner,),
                device_id_type=pl.DeviceIdType.MESH,
            )
            rdma.start()
            rdma.wait()
            acc_ref[pl.ds(keep_base, half), :] = (
                acc_ref[pl.ds(keep_base, half), :] + recv_ref[r, :half]
            )
            base = keep_base

        for step, k in enumerate(reversed(range(LOG2_N))):
            blk = ROWS >> (k + 1)
            slot = LOG2_N + step
            partner = my ^ (1 << k)
            rdma = pltpu.make_async_remote_copy(
                src_ref=acc_ref.at[pl.ds(base, blk)],
                dst_ref=acc_ref.at[pl.ds(base, blk)],
                send_sem=send_sems.at[slot],
                recv_sem=recv_sems.at[slot],
                device_id=(partner,),
                device_id_type=pl.DeviceIdType.MESH,
            )
            rdma.start()
            rdma.wait()
            base = base - ((my >> k) & 1) * blk

        out_ref[...] = acc_ref[...].reshape(B, SQ, DM)

    return pl.pallas_call(
        body,
        out_shape=jax.ShapeDtypeStruct((B, SQ, DM), jnp.float32),
        in_specs=[
            pl.BlockSpec(memory_space=pltpu.VMEM),
            pl.BlockSpec(memory_space=pltpu.VMEM),
            pl.BlockSpec(memory_space=pltpu.VMEM),
            pl.BlockSpec(memory_space=pltpu.VMEM),
            pl.BlockSpec(memory_space=pltpu.VMEM),
        ],
        out_specs=pl.BlockSpec(memory_space=pltpu.VMEM),
        scratch_shapes=[
            pltpu.VMEM((ROWS, DM), jnp.float32),
            pltpu.VMEM((LOG2_N, ROWS // 2, DM), jnp.float32),
            pltpu.SemaphoreType.DMA((2 * LOG2_N,)),
            pltpu.SemaphoreType.DMA((2 * LOG2_N,)),
        ],
    )(x, Wq, k_loc, v_loc, Wo)
